# baseline (speedup 1.0000x reference)
import functools

import jax
import jax.numpy as jnp
from jax import lax
from jax.experimental import pallas as pl
from jax.experimental.pallas import tpu as pltpu
from jax.experimental.pallas import tpu_sc as plsc

_INFO = plsc.get_sparse_core_info()
_NC = _INFO.num_cores
_NS = _INFO.num_subcores
_NW = _NC * _NS


def _make_sc_gather(V, D, B):
    assert B % _NW == 0
    b_per_w = B // _NW
    mesh = plsc.VectorSubcoreMesh(core_axis_name="c", subcore_axis_name="s")

    @functools.partial(
        pl.kernel,
        mesh=mesh,
        out_type=(
            jax.ShapeDtypeStruct((B, D), jnp.float32),
            jax.ShapeDtypeStruct((B, D), jnp.float32),
        ),
        scratch_types=[
            pltpu.VMEM((b_per_w,), jnp.int32),
            pltpu.VMEM((b_per_w, D), jnp.float32),
            pltpu.SemaphoreType.DMA,
        ],
    )
    def sc_gather(table_hbm, idx1_hbm, idx2_hbm, e1_hbm, e2_hbm,
                  idx_v, rows_v, sem):
        wid = lax.axis_index("s") * _NC + lax.axis_index("c")
        base = wid * b_per_w
        pltpu.sync_copy(idx1_hbm.at[pl.ds(base, b_per_w)], idx_v)
        pltpu.async_copy(table_hbm.at[idx_v], rows_v, sem).wait()
        pltpu.sync_copy(rows_v, e1_hbm.at[pl.ds(base, b_per_w)])
        pltpu.sync_copy(idx2_hbm.at[pl.ds(base, b_per_w)], idx_v)
        pltpu.async_copy(table_hbm.at[idx_v], rows_v, sem).wait()
        pltpu.sync_copy(rows_v, e2_hbm.at[pl.ds(base, b_per_w)])

    return sc_gather


_NCHUNK = 16


def _concat_dma_body(alpha_ref, e1_ref, e2_ref, feat_ref, out_ref,
                     emb_ref, feat_sems, emb_sem):
    B, T1, D = out_ref.shape
    T = T1 - 1
    chunk = B // _NCHUNK
    for c in range(_NCHUNK):
        pltpu.make_async_copy(
            feat_ref.at[pl.ds(c * chunk, chunk)],
            out_ref.at[pl.ds(c * chunk, chunk), pl.ds(1, T)],
            feat_sems.at[c],
        ).start()
    a = alpha_ref[0, 0]
    emb_ref[:, 0, :] = a * e1_ref[...] + (1.0 - a) * e2_ref[...]
    pltpu.make_async_copy(
        emb_ref, out_ref.at[:, pl.ds(0, 1)], emb_sem,
    ).start()
    pltpu.make_async_copy(
        emb_ref, out_ref.at[:, pl.ds(0, 1)], emb_sem,
    ).wait()
    for c in range(_NCHUNK):
        pltpu.make_async_copy(
            feat_ref.at[pl.ds(c * chunk, chunk)],
            out_ref.at[pl.ds(c * chunk, chunk), pl.ds(1, T)],
            feat_sems.at[c],
        ).wait()


def kernel(feature, index_value_1, index_value_2, embedding_table, alpha):
    B, T, D = feature.shape
    V = embedding_table.shape[0]
    idx1 = index_value_1.astype(jnp.int32)
    idx2 = index_value_2.astype(jnp.int32)

    e1, e2 = _make_sc_gather(V, D, B)(embedding_table, idx1, idx2)

    out = pl.pallas_call(
        _concat_dma_body,
        in_specs=[
            pl.BlockSpec(memory_space=pltpu.SMEM),
            pl.BlockSpec(memory_space=pltpu.VMEM),
            pl.BlockSpec(memory_space=pltpu.VMEM),
            pl.BlockSpec(memory_space=pltpu.MemorySpace.HBM),
        ],
        out_specs=pl.BlockSpec(memory_space=pltpu.MemorySpace.HBM),
        out_shape=jax.ShapeDtypeStruct((B, T + 1, D), jnp.float32),
        scratch_shapes=[
            pltpu.VMEM((B, 1, D), jnp.float32),
            pltpu.SemaphoreType.DMA((_NCHUNK,)),
            pltpu.SemaphoreType.DMA,
        ],
    )(jnp.reshape(alpha.astype(jnp.float32), (1, 1)), e1, e2, feature)
    return out


# speedup vs baseline: 20.9440x; 20.9440x over previous
import functools

import jax
import jax.numpy as jnp
from jax import lax
from jax.experimental import pallas as pl
from jax.experimental.pallas import tpu as pltpu
from jax.experimental.pallas import tpu_sc as plsc

_INFO = plsc.get_sparse_core_info()
_NC = _INFO.num_cores
_NS = _INFO.num_subcores
_NW = _NC * _NS


def _make_sc_gather(V, D, B):
    assert B % _NW == 0
    b_per_w = B // _NW
    mesh = plsc.VectorSubcoreMesh(core_axis_name="c", subcore_axis_name="s")

    @functools.partial(
        pl.kernel,
        mesh=mesh,
        out_type=(
            jax.ShapeDtypeStruct((B, D), jnp.float32),
            jax.ShapeDtypeStruct((B, D), jnp.float32),
        ),
        scratch_types=[
            pltpu.VMEM((b_per_w,), jnp.int32),
            pltpu.VMEM((b_per_w, D), jnp.float32),
            pltpu.SemaphoreType.DMA,
        ],
    )
    def sc_gather(table_hbm, idx1_hbm, idx2_hbm, e1_hbm, e2_hbm,
                  idx_v, rows_v, sem):
        wid = lax.axis_index("s") * _NC + lax.axis_index("c")
        base = wid * b_per_w
        pltpu.sync_copy(idx1_hbm.at[pl.ds(base, b_per_w)], idx_v)
        pltpu.async_copy(table_hbm.at[idx_v], rows_v, sem).wait()
        pltpu.sync_copy(rows_v, e1_hbm.at[pl.ds(base, b_per_w)])
        pltpu.sync_copy(idx2_hbm.at[pl.ds(base, b_per_w)], idx_v)
        pltpu.async_copy(table_hbm.at[idx_v], rows_v, sem).wait()
        pltpu.sync_copy(rows_v, e2_hbm.at[pl.ds(base, b_per_w)])

    return sc_gather


_BB = 32
_NBUF = 8
_AHEAD = 4


def _concat_ring_body(alpha_ref, e1_ref, e2_ref, feat_ref, out_ref,
                      bufs, in_sems, out_sems):
    B, T1, D = out_ref.shape
    T = T1 - 1
    nblk = B // _BB
    a = alpha_ref[0, 0]

    def start_in(g):
        s = g % _NBUF
        pltpu.make_async_copy(
            feat_ref.at[pl.ds(g * _BB, _BB)],
            bufs.at[s, :, pl.ds(1, T), :],
            in_sems.at[s],
        ).start()

    def wait_in(g):
        s = g % _NBUF
        pltpu.make_async_copy(
            feat_ref.at[pl.ds(g * _BB, _BB)],
            bufs.at[s, :, pl.ds(1, T), :],
            in_sems.at[s],
        ).wait()

    def start_out(g):
        s = g % _NBUF
        pltpu.make_async_copy(
            bufs.at[s], out_ref.at[pl.ds(g * _BB, _BB)], out_sems.at[s],
        ).start()

    def wait_out(g):
        s = g % _NBUF
        pltpu.make_async_copy(
            bufs.at[s], out_ref.at[pl.ds(g * _BB, _BB)], out_sems.at[s],
        ).wait()

    for g in range(_AHEAD):
        start_in(g)
    for g in range(nblk):
        nxt = g + _AHEAD
        if nxt < nblk:
            if nxt >= _NBUF:
                wait_out(nxt - _NBUF)
            start_in(nxt)
        wait_in(g)
        s = g % _NBUF
        eb = a * e1_ref[pl.ds(g * _BB, _BB), :] \
            + (1.0 - a) * e2_ref[pl.ds(g * _BB, _BB), :]
        bufs[s, :, 0:1, :] = eb[:, None, :]
        start_out(g)
    for g in range(max(nblk - _NBUF, 0), nblk):
        wait_out(g)


def kernel(feature, index_value_1, index_value_2, embedding_table, alpha):
    B, T, D = feature.shape
    V = embedding_table.shape[0]
    idx1 = index_value_1.astype(jnp.int32)
    idx2 = index_value_2.astype(jnp.int32)

    e1, e2 = _make_sc_gather(V, D, B)(embedding_table, idx1, idx2)

    out = pl.pallas_call(
        _concat_ring_body,
        in_specs=[
            pl.BlockSpec(memory_space=pltpu.SMEM),
            pl.BlockSpec(memory_space=pltpu.VMEM),
            pl.BlockSpec(memory_space=pltpu.VMEM),
            pl.BlockSpec(memory_space=pltpu.MemorySpace.HBM),
        ],
        out_specs=pl.BlockSpec(memory_space=pltpu.MemorySpace.HBM),
        out_shape=jax.ShapeDtypeStruct((B, T + 1, D), jnp.float32),
        scratch_shapes=[
            pltpu.VMEM((_NBUF, _BB, T + 1, D), jnp.float32),
            pltpu.SemaphoreType.DMA((_NBUF,)),
            pltpu.SemaphoreType.DMA((_NBUF,)),
        ],
    )(jnp.reshape(alpha.astype(jnp.float32), (1, 1)), e1, e2, feature)
    return out
